# baseline (speedup 1.0000x reference)
import functools

import jax
import jax.numpy as jnp
from jax import lax
from jax.experimental import pallas as pl
from jax.experimental.pallas import tpu as pltpu
from jax.experimental.pallas import tpu_sc as plsc

N = 10000
E = 320000
D = 128
NC, NS = 2, 16
NW = NC * NS
E_PAD = 327680
CHUNK = E_PAD // NW
B = 96
SEC = 1280
CAP = CHUNK + 2 * B
FT_ROWS = 10112
HIT_ROWS = 10240
DUMP = N
SHIFT = 14


def _sc_scatter(feat, src, dst, dep, dep16, zft, zhit):
    mesh = plsc.VectorSubcoreMesh(
        core_axis_name="c", subcore_axis_name="s",
        num_cores=NC, num_subcores=NS)

    @functools.partial(
        pl.kernel,
        out_type=(
            jax.ShapeDtypeStruct((NC, N, D), jnp.float32),
            jax.ShapeDtypeStruct((NC, HIT_ROWS), jnp.float32),
        ),
        mesh=mesh,
        scratch_types=[
            pltpu.VMEM((SEC,), jnp.int32),
            pltpu.VMEM((SEC,), jnp.int32),
            pltpu.VMEM((SEC,), jnp.int32),
            pltpu.VMEM((CAP,), jnp.int32),
            pltpu.VMEM((B, D), jnp.float32),
            pltpu.VMEM((B, D), jnp.float32),
            pltpu.VMEM((B,), jnp.int32),
            pltpu.VMEM((B,), jnp.int32),
            pltpu.VMEM((B,), jnp.int32),
            pltpu.VMEM((B,), jnp.int32),
            pltpu.VMEM((B,), jnp.float32),
            pltpu.VMEM((16,), jnp.int32),
            pltpu.VMEM_SHARED((FT_ROWS, D), jnp.float32),
            pltpu.VMEM_SHARED((HIT_ROWS,), jnp.float32),
            pltpu.SemaphoreType.DMA,
            pltpu.SemaphoreType.DMA,
        ],
    )
    def k(feat_h, src_h, dst_h, dep_h, dep16_h, zft_h, zhit_h,
          ftout_h, hitout_h,
          src_s, dst_s, dep_s, cdst, rows0, rows1,
          srcb0, srcb1, dstb0, dstb1, ones, dep16_v,
          ft_acc, hit_acc, sem0, sem1):
        cid = lax.axis_index("c")
        sid = lax.axis_index("s")
        wid = sid * NC + cid
        base = wid * CHUNK

        pltpu.sync_copy(dep16_h, dep16_v)

        zr = FT_ROWS // NS
        pltpu.sync_copy(zft_h.at[pl.ds(sid * zr, zr)],
                        ft_acc.at[pl.ds(sid * zr, zr)])
        hr = HIT_ROWS // NS
        pltpu.sync_copy(zhit_h.at[pl.ds(sid * hr, hr)],
                        hit_acc.at[pl.ds(sid * hr, hr)])

        def fill_ones(j, carry):
            ones[pl.ds(j * 16, 16)] = jnp.ones((16,), jnp.float32)
            return carry
        lax.fori_loop(0, B // 16, fill_ones, 0)

        dvs = dep16_v[pl.ds(0, 16)][0]

        def section(s, ptr):
            pltpu.sync_copy(src_h.at[pl.ds(base + s * SEC, SEC)], src_s)
            pltpu.sync_copy(dst_h.at[pl.ds(base + s * SEC, SEC)], dst_s)
            pltpu.sync_copy(dep_h.at[pl.ds(base + s * SEC, SEC)], dep_s)

            def compact(i, p):
                d16 = dep_s[pl.ds(i * 16, 16)]
                s16 = src_s[pl.ds(i * 16, 16)]
                t16 = dst_s[pl.ds(i * 16, 16)]
                q = p
                for j in range(16):
                    pkj = s16[j] | (t16[j] << SHIFT)
                    cdst[pl.ds(q, 16)] = jnp.full((16,), pkj, jnp.int32)
                    q = q + jnp.where(d16[j] == dvs, 1, 0)
                return q
            return lax.fori_loop(0, SEC // 16, compact, ptr)
        ptr = lax.fori_loop(0, CHUNK // SEC, section, 0)

        def pad_tail(j, carry):
            cdst[pl.ds(ptr + j * 16, 16)] = jnp.full(
                (16,), DUMP << SHIFT, jnp.int32)
            return carry
        lax.fori_loop(0, 2 * B // 16, pad_tail, 0)

        nb = (ptr + B - 1) // B
        nb2 = (nb + 1) // 2

        plsc.subcore_barrier()

        def unpack_batch(b, sref, dref):
            def one(v, carry):
                pk = cdst[pl.ds(b * B + v * 16, 16)]
                sref[pl.ds(v * 16, 16)] = pk & ((1 << SHIFT) - 1)
                dref[pl.ds(v * 16, 16)] = pk >> SHIFT
                return carry
            lax.fori_loop(0, B // 16, one, 0)

        @pl.when(nb2 > 0)
        def _():
            unpack_batch(0, srcb0, dstb0)
            pltpu.async_copy(feat_h.at[srcb0], rows0, sem0)

        def dbatch(g, carry):
            b0 = 2 * g
            b1 = 2 * g + 1
            unpack_batch(b1, srcb1, dstb1)
            pltpu.async_copy(feat_h.at[srcb1], rows1, sem1)
            pltpu.make_async_copy(feat_h.at[srcb0], rows0, sem0).wait()
            pltpu.sync_copy(rows0, ft_acc.at[dstb0], add=True)
            pltpu.sync_copy(ones, hit_acc.at[dstb0], add=True)

            @pl.when(b0 + 2 < 2 * nb2)
            def _():
                unpack_batch(b0 + 2, srcb0, dstb0)
                pltpu.async_copy(feat_h.at[srcb0], rows0, sem0)
            pltpu.make_async_copy(feat_h.at[srcb1], rows1, sem1).wait()
            pltpu.sync_copy(rows1, ft_acc.at[dstb1], add=True)
            pltpu.sync_copy(ones, hit_acc.at[dstb1], add=True)
            return carry
        lax.fori_loop(0, nb2, dbatch, 0)

        plsc.subcore_barrier()

        @pl.when(sid < 10)
        def _():
            r0 = sid * 1000
            pltpu.sync_copy(ft_acc.at[pl.ds(r0, 1000)],
                            ftout_h.at[cid].at[pl.ds(r0, 1000)])
        h0 = sid * hr
        pltpu.sync_copy(hit_acc.at[pl.ds(h0, hr)],
                        hitout_h.at[cid].at[pl.ds(h0, hr)])

    return k(feat, src, dst, dep, dep16, zft, zhit)


def _tc_combine(feat, ft0, ft1, hit0, hit1, W1, W2, b2):
    R = 400

    def body(feat_r, ft0_r, ft1_r, hit0_r, hit1_r, W1_r, W2_r, b_r, out_r):
        f = feat_r[...]
        ft = ft0_r[...] + ft1_r[...]
        res = jnp.dot(f, W1_r[...], preferred_element_type=jnp.float32)
        res = res + jnp.dot(ft, W2_r[...], preferred_element_type=jnp.float32)
        res = jnp.maximum(res + b_r[...], 0.0)
        hit = hit0_r[...] + hit1_r[...]
        out_r[...] = jnp.where(hit > 0.0, res, f)

    return pl.pallas_call(
        body,
        grid=(N // R,),
        in_specs=[
            pl.BlockSpec((R, D), lambda i: (i, 0)),
            pl.BlockSpec((R, D), lambda i: (i, 0)),
            pl.BlockSpec((R, D), lambda i: (i, 0)),
            pl.BlockSpec((R, 1), lambda i: (i, 0)),
            pl.BlockSpec((R, 1), lambda i: (i, 0)),
            pl.BlockSpec((D, D), lambda i: (0, 0)),
            pl.BlockSpec((D, D), lambda i: (0, 0)),
            pl.BlockSpec((1, D), lambda i: (0, 0)),
        ],
        out_specs=pl.BlockSpec((R, D), lambda i: (i, 0)),
        out_shape=jax.ShapeDtypeStruct((N, D), jnp.float32),
    )(feat, ft0, ft1, hit0, hit1, W1, W2, b2)


def kernel(feat, edge_index, edge_depth, depth, W, b):
    pad = E_PAD - E
    src = jnp.concatenate([edge_index[0], jnp.zeros((pad,), jnp.int32)])
    dst = jnp.concatenate([edge_index[1], jnp.full((pad,), DUMP, jnp.int32)])
    edge_depth = jnp.concatenate(
        [edge_depth, jnp.full((pad,), -1, jnp.int32)])
    dep16 = jnp.full((16,), depth, jnp.int32)
    zft = jnp.zeros((FT_ROWS, D), jnp.float32)
    zhit = jnp.zeros((HIT_ROWS,), jnp.float32)
    ftp, hitp = _sc_scatter(feat, src, dst, edge_depth, dep16, zft, zhit)
    hitp = hitp[:, :N]
    W1 = W[:D]
    W2 = W[D:]
    b2 = b.reshape(1, D)
    hit0 = hitp[0].reshape(N, 1)
    hit1 = hitp[1].reshape(N, 1)
    return _tc_combine(feat, ftp[0], ftp[1], hit0, hit1, W1, W2, b2)


# speedup vs baseline: 1.3226x; 1.3226x over previous
import functools

import jax
import jax.numpy as jnp
from jax import lax
from jax.experimental import pallas as pl
from jax.experimental.pallas import tpu as pltpu
from jax.experimental.pallas import tpu_sc as plsc

N = 10000
E = 320000
D = 128
NC, NS = 2, 16
NW = NC * NS
E_PAD = 327680
CHUNK = E_PAD // NW
B = 64
SEC = 1280
CAP = CHUNK + 2 * B
FT_ROWS = 10112
HIT_ROWS = 10240
DUMP = N
SHIFT = 14


def _sc_scatter(feat, src, dst, dep, dep16):
    mesh = plsc.VectorSubcoreMesh(
        core_axis_name="c", subcore_axis_name="s",
        num_cores=NC, num_subcores=NS)

    @functools.partial(
        pl.kernel,
        out_type=(
            jax.ShapeDtypeStruct((NC, N, D), jnp.float32),
            jax.ShapeDtypeStruct((NC, HIT_ROWS), jnp.float32),
        ),
        mesh=mesh,
        scratch_types=[
            pltpu.VMEM((SEC,), jnp.int32),
            pltpu.VMEM((SEC,), jnp.int32),
            pltpu.VMEM((SEC,), jnp.int32),
            pltpu.VMEM((CAP,), jnp.int32),
            pltpu.VMEM((CAP,), jnp.int32),
            pltpu.VMEM((B, D), jnp.float32),
            pltpu.VMEM((B, D), jnp.float32),
            pltpu.VMEM((2 * B,), jnp.float32),
            pltpu.VMEM((HIT_ROWS // NS,), jnp.float32),
            pltpu.VMEM((16,), jnp.int32),
            pltpu.VMEM_SHARED((FT_ROWS, D), jnp.float32),
            pltpu.VMEM_SHARED((HIT_ROWS,), jnp.float32),
            pltpu.SemaphoreType.DMA,
            pltpu.SemaphoreType.DMA,
            pltpu.SemaphoreType.DMA,
        ],
    )
    def k(feat_h, src_h, dst_h, dep_h, dep16_h,
          ftout_h, hitout_h,
          src_s, dst_s, dep_s, csrc, cdst, rows0, rows1, ones, zbuf,
          dep16_v, ft_acc, hit_acc, sem0, sem1, semh):
        cid = lax.axis_index("c")
        sid = lax.axis_index("s")
        wid = sid * NC + cid
        base = wid * CHUNK

        pltpu.sync_copy(dep16_h, dep16_v)

        def zfill_rows(i, carry):
            rows0[i // 8, pl.ds((i % 8) * 16, 16)] = jnp.zeros(
                (16,), jnp.float32)
            return carry
        lax.fori_loop(0, B * D // 16, zfill_rows, 0)

        hr = HIT_ROWS // NS

        def zfill_hit(j, carry):
            zbuf[pl.ds(j * 16, 16)] = jnp.zeros((16,), jnp.float32)
            return carry
        lax.fori_loop(0, hr // 16, zfill_hit, 0)

        zr = FT_ROWS // NS
        r0z = sid * zr

        def zcopy(i, carry):
            pltpu.sync_copy(rows0, ft_acc.at[pl.ds(r0z + i * B, B)])
            return carry
        lax.fori_loop(0, zr // B, zcopy, 0)
        pltpu.sync_copy(rows0.at[pl.ds(0, zr % B)],
                        ft_acc.at[pl.ds(r0z + (zr // B) * B, zr % B)])
        pltpu.sync_copy(zbuf, hit_acc.at[pl.ds(sid * hr, hr)])

        def fill_ones(j, carry):
            ones[pl.ds(j * 16, 16)] = jnp.ones((16,), jnp.float32)
            return carry
        lax.fori_loop(0, 2 * B // 16, fill_ones, 0)

        dvs = dep16_v[pl.ds(0, 16)][0]

        def section(s, ptr):
            pltpu.sync_copy(src_h.at[pl.ds(base + s * SEC, SEC)], src_s)
            pltpu.sync_copy(dst_h.at[pl.ds(base + s * SEC, SEC)], dst_s)
            pltpu.sync_copy(dep_h.at[pl.ds(base + s * SEC, SEC)], dep_s)

            def compact(i, p):
                d16 = dep_s[pl.ds(i * 16, 16)]
                s16 = src_s[pl.ds(i * 16, 16)]
                t16 = dst_s[pl.ds(i * 16, 16)]
                q = p
                for j in range(16):
                    pkj = s16[j] | (t16[j] << SHIFT)
                    cdst[pl.ds(q, 16)] = jnp.full((16,), pkj, jnp.int32)
                    q = q + jnp.where(d16[j] == dvs, 1, 0)
                return q
            return lax.fori_loop(0, SEC // 16, compact, ptr)
        ptr = lax.fori_loop(0, CHUNK // SEC, section, 0)

        def pad_tail(j, carry):
            cdst[pl.ds(ptr + j * 16, 16)] = jnp.full(
                (16,), DUMP << SHIFT, jnp.int32)
            return carry
        lax.fori_loop(0, 2 * B // 16, pad_tail, 0)

        nb = (ptr + B - 1) // B
        nb2 = (nb + 1) // 2

        def unpack(i, carry):
            pk = cdst[pl.ds(i * 16, 16)]
            csrc[pl.ds(i * 16, 16)] = pk & ((1 << SHIFT) - 1)
            cdst[pl.ds(i * 16, 16)] = pk >> SHIFT
            return carry
        lax.fori_loop(0, nb2 * (2 * B // 16), unpack, 0)

        plsc.subcore_barrier()

        @pl.when(nb2 > 0)
        def _():
            pltpu.async_copy(feat_h.at[csrc.at[pl.ds(0, B)]], rows0, sem0)

        def dbatch(g, carry):
            b0 = 2 * g
            b1 = 2 * g + 1
            pltpu.async_copy(feat_h.at[csrc.at[pl.ds(b1 * B, B)]], rows1,
                             sem1)
            pltpu.async_copy(ones,
                             hit_acc.at[cdst.at[pl.ds(b0 * B, 2 * B)]],
                             semh, add=True)
            pltpu.make_async_copy(feat_h.at[csrc.at[pl.ds(b0 * B, B)]],
                                  rows0, sem0).wait()
            pltpu.sync_copy(rows0, ft_acc.at[cdst.at[pl.ds(b0 * B, B)]],
                            add=True)

            @pl.when(b0 + 2 < 2 * nb2)
            def _():
                pltpu.async_copy(feat_h.at[csrc.at[pl.ds((b0 + 2) * B, B)]],
                                 rows0, sem0)
            pltpu.make_async_copy(feat_h.at[csrc.at[pl.ds(b1 * B, B)]],
                                  rows1, sem1).wait()
            pltpu.sync_copy(rows1, ft_acc.at[cdst.at[pl.ds(b1 * B, B)]],
                            add=True)
            return carry
        lax.fori_loop(0, nb2, dbatch, 0)

        def drain_hits(g, carry):
            pltpu.make_async_copy(
                ones, hit_acc.at[cdst.at[pl.ds(0, 2 * B)]], semh).wait()
            return carry
        lax.fori_loop(0, nb2, drain_hits, 0)

        plsc.subcore_barrier()

        @pl.when(sid < 10)
        def _():
            r0 = sid * 1000
            pltpu.sync_copy(ft_acc.at[pl.ds(r0, 1000)],
                            ftout_h.at[cid].at[pl.ds(r0, 1000)])
        h0 = sid * hr
        pltpu.sync_copy(hit_acc.at[pl.ds(h0, hr)],
                        hitout_h.at[cid].at[pl.ds(h0, hr)])

    return k(feat, src, dst, dep, dep16)


def _tc_combine(feat, ftp, hitp3, W1, W2, b2):
    R = 400

    def body(feat_r, ft0_r, ft1_r, hit0_r, hit1_r, W1_r, W2_r, b_r, out_r):
        f = feat_r[...]
        ft = ft0_r[...][0] + ft1_r[...][0]
        res = jnp.dot(f, W1_r[...], preferred_element_type=jnp.float32)
        res = res + jnp.dot(ft, W2_r[...], preferred_element_type=jnp.float32)
        res = jnp.maximum(res + b_r[...], 0.0)
        hit = hit0_r[...][0] + hit1_r[...][0]
        out_r[...] = jnp.where(hit > 0.0, res, f)

    return pl.pallas_call(
        body,
        grid=(N // R,),
        in_specs=[
            pl.BlockSpec((R, D), lambda i: (i, 0)),
            pl.BlockSpec((1, R, D), lambda i: (0, i, 0)),
            pl.BlockSpec((1, R, D), lambda i: (1, i, 0)),
            pl.BlockSpec((1, R, 1), lambda i: (0, i, 0)),
            pl.BlockSpec((1, R, 1), lambda i: (1, i, 0)),
            pl.BlockSpec((D, D), lambda i: (0, 0)),
            pl.BlockSpec((D, D), lambda i: (0, 0)),
            pl.BlockSpec((1, D), lambda i: (0, 0)),
        ],
        out_specs=pl.BlockSpec((R, D), lambda i: (i, 0)),
        out_shape=jax.ShapeDtypeStruct((N, D), jnp.float32),
    )(feat, ftp, ftp, hitp3, hitp3, W1, W2, b2)


def kernel(feat, edge_index, edge_depth, depth, W, b):
    pad = E_PAD - E
    src = jnp.concatenate([edge_index[0], jnp.zeros((pad,), jnp.int32)])
    dst = jnp.concatenate([edge_index[1], jnp.full((pad,), DUMP, jnp.int32)])
    edge_depth = jnp.concatenate(
        [edge_depth, jnp.full((pad,), -1, jnp.int32)])
    dep16 = jnp.full((16,), depth, jnp.int32)
    ftp, hitp = _sc_scatter(feat, src, dst, edge_depth, dep16)
    hitp3 = hitp.reshape(NC, HIT_ROWS, 1)
    W1 = W[:D]
    W2 = W[D:]
    b2 = b.reshape(1, D)
    return _tc_combine(feat, ftp, hitp3, W1, W2, b2)


# speedup vs baseline: 1.3917x; 1.0523x over previous
import functools

import jax
import jax.numpy as jnp
from jax import lax
from jax.experimental import pallas as pl
from jax.experimental.pallas import tpu as pltpu
from jax.experimental.pallas import tpu_sc as plsc

N = 10000
E = 320000
D = 128
NC, NS = 2, 16
NW = NC * NS
E_PAD = 327680
CHUNK = E_PAD // NW
B = 64
SEC = 1280
CAP = CHUNK + 2 * B
FT_ROWS = 10112
HIT_ROWS = 10240
DUMP = N
SHIFT = 14


def _sc_scatter(feat, src, dst, dep, dep16):
    mesh = plsc.VectorSubcoreMesh(
        core_axis_name="c", subcore_axis_name="s",
        num_cores=NC, num_subcores=NS)

    @functools.partial(
        pl.kernel,
        out_type=(
            jax.ShapeDtypeStruct((NC, N, D), jnp.float32),
            jax.ShapeDtypeStruct((NC, HIT_ROWS), jnp.float32),
        ),
        mesh=mesh,
        scratch_types=[
            pltpu.VMEM((SEC,), jnp.int32),
            pltpu.VMEM((SEC,), jnp.int32),
            pltpu.VMEM((SEC,), jnp.int32),
            pltpu.VMEM((CAP,), jnp.int32),
            pltpu.VMEM((CAP,), jnp.int32),
            pltpu.VMEM((B, D), jnp.float32),
            pltpu.VMEM((B, D), jnp.float32),
            pltpu.VMEM((2 * B,), jnp.float32),
            pltpu.VMEM((HIT_ROWS // NS,), jnp.float32),
            pltpu.VMEM((16,), jnp.int32),
            pltpu.VMEM_SHARED((FT_ROWS, D), jnp.float32),
            pltpu.VMEM_SHARED((HIT_ROWS,), jnp.float32),
            pltpu.SemaphoreType.DMA,
            pltpu.SemaphoreType.DMA,
            pltpu.SemaphoreType.DMA,
            pltpu.SemaphoreType.DMA,
            pltpu.SemaphoreType.DMA,
        ],
    )
    def k(feat_h, src_h, dst_h, dep_h, dep16_h,
          ftout_h, hitout_h,
          src_s, dst_s, dep_s, csrc, cdst, rows0, rows1, ones, zbuf,
          dep16_v, ft_acc, hit_acc, sem0, sem1, semh, semz, sems):
        cid = lax.axis_index("c")
        sid = lax.axis_index("s")
        wid = sid * NC + cid
        base = wid * CHUNK

        pltpu.sync_copy(dep16_h, dep16_v)

        def zfill_rows(i, carry):
            rows0[i // 8, pl.ds((i % 8) * 16, 16)] = jnp.zeros(
                (16,), jnp.float32)
            return carry
        lax.fori_loop(0, B * D // 16, zfill_rows, 0)

        hr = HIT_ROWS // NS

        def zfill_hit(j, carry):
            zbuf[pl.ds(j * 16, 16)] = jnp.zeros((16,), jnp.float32)
            return carry
        lax.fori_loop(0, hr // 16, zfill_hit, 0)

        zr = FT_ROWS // NS
        r0z = sid * zr

        def zcopy(i, carry):
            pltpu.async_copy(rows0, ft_acc.at[pl.ds(r0z + i * B, B)], semz)
            return carry
        lax.fori_loop(0, zr // B, zcopy, 0)
        pltpu.async_copy(rows0.at[pl.ds(0, zr % B)],
                         ft_acc.at[pl.ds(r0z + (zr // B) * B, zr % B)], semz)
        pltpu.async_copy(zbuf, hit_acc.at[pl.ds(sid * hr, hr)], semz)

        def fill_ones(j, carry):
            ones[pl.ds(j * 16, 16)] = jnp.ones((16,), jnp.float32)
            return carry
        lax.fori_loop(0, 2 * B // 16, fill_ones, 0)

        dvs = dep16_v[pl.ds(0, 16)][0]

        def section(s, ptr):
            pltpu.async_copy(src_h.at[pl.ds(base + s * SEC, SEC)], src_s,
                             sems)
            pltpu.async_copy(dst_h.at[pl.ds(base + s * SEC, SEC)], dst_s,
                             sems)
            pltpu.async_copy(dep_h.at[pl.ds(base + s * SEC, SEC)], dep_s,
                             sems)
            pltpu.make_async_copy(src_h.at[pl.ds(base + s * SEC, SEC)],
                                  src_s, sems).wait()
            pltpu.make_async_copy(dst_h.at[pl.ds(base + s * SEC, SEC)],
                                  dst_s, sems).wait()
            pltpu.make_async_copy(dep_h.at[pl.ds(base + s * SEC, SEC)],
                                  dep_s, sems).wait()

            def compact(i, p):
                d16 = dep_s[pl.ds(i * 16, 16)]
                s16 = src_s[pl.ds(i * 16, 16)]
                t16 = dst_s[pl.ds(i * 16, 16)]
                q = p
                for j in range(16):
                    pkj = s16[j] | (t16[j] << SHIFT)
                    cdst[pl.ds(q, 16)] = jnp.full((16,), pkj, jnp.int32)
                    q = q + jnp.where(d16[j] == dvs, 1, 0)
                return q
            return lax.fori_loop(0, SEC // 16, compact, ptr)
        ptr = lax.fori_loop(0, CHUNK // SEC, section, 0)

        def pad_tail(j, carry):
            cdst[pl.ds(ptr + j * 16, 16)] = jnp.full(
                (16,), DUMP << SHIFT, jnp.int32)
            return carry
        lax.fori_loop(0, 2 * B // 16, pad_tail, 0)

        nb = (ptr + B - 1) // B
        nb2 = (nb + 1) // 2

        def unpack(i, carry):
            pk = cdst[pl.ds(i * 16, 16)]
            csrc[pl.ds(i * 16, 16)] = pk & ((1 << SHIFT) - 1)
            cdst[pl.ds(i * 16, 16)] = pk >> SHIFT
            return carry
        lax.fori_loop(0, nb2 * (2 * B // 16), unpack, 0)

        def zdrain(i, carry):
            pltpu.make_async_copy(rows0, ft_acc.at[pl.ds(r0z, B)],
                                  semz).wait()
            return carry
        lax.fori_loop(0, zr // B, zdrain, 0)
        pltpu.make_async_copy(rows0.at[pl.ds(0, zr % B)],
                              ft_acc.at[pl.ds(r0z, zr % B)], semz).wait()
        pltpu.make_async_copy(zbuf, hit_acc.at[pl.ds(0, hr)], semz).wait()

        plsc.subcore_barrier()

        @pl.when(nb2 > 0)
        def _():
            pltpu.async_copy(feat_h.at[csrc.at[pl.ds(0, B)]], rows0, sem0)

        def dbatch(g, carry):
            b0 = 2 * g
            b1 = 2 * g + 1
            pltpu.async_copy(feat_h.at[csrc.at[pl.ds(b1 * B, B)]], rows1,
                             sem1)
            pltpu.async_copy(ones,
                             hit_acc.at[cdst.at[pl.ds(b0 * B, 2 * B)]],
                             semh, add=True)
            pltpu.make_async_copy(feat_h.at[csrc.at[pl.ds(b0 * B, B)]],
                                  rows0, sem0).wait()
            pltpu.sync_copy(rows0, ft_acc.at[cdst.at[pl.ds(b0 * B, B)]],
                            add=True)

            @pl.when(b0 + 2 < 2 * nb2)
            def _():
                pltpu.async_copy(feat_h.at[csrc.at[pl.ds((b0 + 2) * B, B)]],
                                 rows0, sem0)
            pltpu.make_async_copy(feat_h.at[csrc.at[pl.ds(b1 * B, B)]],
                                  rows1, sem1).wait()
            pltpu.sync_copy(rows1, ft_acc.at[cdst.at[pl.ds(b1 * B, B)]],
                            add=True)
            return carry
        lax.fori_loop(0, nb2, dbatch, 0)

        def drain_hits(g, carry):
            pltpu.make_async_copy(
                ones, hit_acc.at[cdst.at[pl.ds(0, 2 * B)]], semh).wait()
            return carry
        lax.fori_loop(0, nb2, drain_hits, 0)

        plsc.subcore_barrier()

        @pl.when(sid < 10)
        def _():
            r0 = sid * 1000
            pltpu.sync_copy(ft_acc.at[pl.ds(r0, 1000)],
                            ftout_h.at[cid].at[pl.ds(r0, 1000)])
        h0 = sid * hr
        pltpu.sync_copy(hit_acc.at[pl.ds(h0, hr)],
                        hitout_h.at[cid].at[pl.ds(h0, hr)])

    return k(feat, src, dst, dep, dep16)


def _tc_pre(feat, W1, b2):
    R = 400

    def body(feat_r, W1_r, b_r, out_r):
        out_r[...] = jnp.dot(feat_r[...], W1_r[...],
                             preferred_element_type=jnp.float32) + b_r[...]

    return pl.pallas_call(
        body,
        grid=(N // R,),
        in_specs=[
            pl.BlockSpec((R, D), lambda i: (i, 0)),
            pl.BlockSpec((D, D), lambda i: (0, 0)),
            pl.BlockSpec((1, D), lambda i: (0, 0)),
        ],
        out_specs=pl.BlockSpec((R, D), lambda i: (i, 0)),
        out_shape=jax.ShapeDtypeStruct((N, D), jnp.float32),
    )(feat, W1, b2)


def _tc_combine(feat, r1, ftp, hitp3, W2):
    R = 400

    def body(feat_r, r1_r, ft0_r, ft1_r, hit0_r, hit1_r, W2_r, out_r):
        f = feat_r[...]
        ft = ft0_r[...][0] + ft1_r[...][0]
        res = r1_r[...] + jnp.dot(ft, W2_r[...],
                                  preferred_element_type=jnp.float32)
        res = jnp.maximum(res, 0.0)
        hit = hit0_r[...][0] + hit1_r[...][0]
        out_r[...] = jnp.where(hit > 0.0, res, f)

    return pl.pallas_call(
        body,
        grid=(N // R,),
        in_specs=[
            pl.BlockSpec((R, D), lambda i: (i, 0)),
            pl.BlockSpec((R, D), lambda i: (i, 0)),
            pl.BlockSpec((1, R, D), lambda i: (0, i, 0)),
            pl.BlockSpec((1, R, D), lambda i: (1, i, 0)),
            pl.BlockSpec((1, R, 1), lambda i: (0, i, 0)),
            pl.BlockSpec((1, R, 1), lambda i: (1, i, 0)),
            pl.BlockSpec((D, D), lambda i: (0, 0)),
        ],
        out_specs=pl.BlockSpec((R, D), lambda i: (i, 0)),
        out_shape=jax.ShapeDtypeStruct((N, D), jnp.float32),
    )(feat, r1, ftp, ftp, hitp3, hitp3, W2)


def kernel(feat, edge_index, edge_depth, depth, W, b):
    pad = E_PAD - E
    src = jnp.concatenate([edge_index[0], jnp.zeros((pad,), jnp.int32)])
    dst = jnp.concatenate([edge_index[1], jnp.full((pad,), DUMP, jnp.int32)])
    edge_depth = jnp.concatenate(
        [edge_depth, jnp.full((pad,), -1, jnp.int32)])
    dep16 = jnp.full((16,), depth, jnp.int32)
    W1 = W[:D]
    W2 = W[D:]
    b2 = b.reshape(1, D)
    ftp, hitp = _sc_scatter(feat, src, dst, edge_depth, dep16)
    r1 = _tc_pre(feat, W1, b2)
    hitp3 = hitp.reshape(NC, HIT_ROWS, 1)
    return _tc_combine(feat, r1, ftp, hitp3, W2)
